# baseline (speedup 1.0000x reference)
import functools

import jax
import jax.numpy as jnp
from jax import lax
from jax.experimental import pallas as pl
from jax.experimental.pallas import tpu as pltpu
from jax.experimental.pallas import tpu_sc as plsc

NC = 2
NS = 16
NW = NC * NS

IDX_W = 128
NBUF = 2
RBUF = 2


def _wid():
    return lax.axis_index("s") * NC + lax.axis_index("c")


@functools.partial(jax.jit, static_argnames=("v",))
def _sc_relayout(e_t, tail2, *, v):
    n_blocks = v // IDX_W
    per_tile = n_blocks // NW
    n_extra = n_blocks - per_tile * NW
    max_nb = per_tile + 1

    @functools.partial(
        pl.kernel,
        mesh=plsc.VectorSubcoreMesh(core_axis_name="c", subcore_axis_name="s"),
        out_type=jax.ShapeDtypeStruct((v // 2, IDX_W), jnp.float32),
        scratch_types=[
            pltpu.VMEM((RBUF, 64, IDX_W), jnp.float32),
            pltpu.VMEM((RBUF, 64, IDX_W), jnp.float32),
            [pltpu.SemaphoreType.DMA] * RBUF,
            [pltpu.SemaphoreType.DMA] * RBUF,
        ],
        compiler_params=pltpu.CompilerParams(needs_layout_passes=False),
    )
    def k(et_hbm, tail_hbm, t2_hbm, inb, outb, rsems, wsems):
        wid = _wid()
        nb_tile = jnp.where(wid < n_extra, per_tile + 1, per_tile)
        rowv = lax.iota(jnp.int32, 16) // 2
        colv = (lax.iota(jnp.int32, 16) % 2) * 64

        def blk_of(g):
            return g * NW + wid

        def fire_read(g, b):
            pltpu.async_copy(
                et_hbm.at[:, pl.ds(blk_of(g) * IDX_W, IDX_W)],
                inb.at[b],
                rsems[b],
            )

        def wait_read(g, b):
            pltpu.make_async_copy(
                et_hbm.at[:, pl.ds(blk_of(g) * IDX_W, IDX_W)],
                inb.at[b],
                rsems[b],
            ).wait()

        def transpose(b):
            for d in range(64):
                for l0 in range(8):
                    vv = inb[b, d, pl.ds(l0 * 16, 16)]
                    plsc.store_scatter(
                        outb.at[b], [rowv + l0 * 8, colv + d], vv
                    )

        def fire_write(g, b):
            pltpu.async_copy(
                outb.at[b],
                t2_hbm.at[pl.ds(blk_of(g) * 64, 64)],
                wsems[b],
            )

        def wait_write(g, b):
            pltpu.make_async_copy(
                outb.at[b],
                t2_hbm.at[pl.ds(blk_of(g) * 64, 64)],
                wsems[b],
            ).wait()

        for b in range(RBUF):
            fire_read(b, b)

        def body(o, carry):
            for b in range(RBUF):
                g = o * RBUF + b

                @pl.when(g < nb_tile)
                def _():
                    wait_read(g, b)

                    @pl.when(g >= RBUF)
                    def _():
                        wait_write(g - RBUF, b)

                    transpose(b)
                    fire_write(g, b)

                    @pl.when(g + RBUF < nb_tile)
                    def _():
                        fire_read(g + RBUF, b)

            return carry

        lax.fori_loop(0, (max_nb + RBUF - 1) // RBUF, body, 0)

        for b in range(RBUF):
            wait_write(0, b)

        @pl.when(wid == n_extra)
        def _():
            pltpu.sync_copy(tail_hbm, inb.at[0, pl.ds(0, 32)])
            pltpu.sync_copy(
                inb.at[0, pl.ds(0, 32)],
                t2_hbm.at[pl.ds(n_blocks * 64, 32)],
            )

    return k(e_t, tail2)


@functools.partial(jax.jit, static_argnames=("n_rows",))
def _sc_gather(tok, t2, *, n_rows):
    rows_per_w = n_rows // NW

    @functools.partial(
        pl.kernel,
        mesh=plsc.VectorSubcoreMesh(core_axis_name="c", subcore_axis_name="s"),
        out_type=jax.ShapeDtypeStruct((n_rows * IDX_W, IDX_W), jnp.float32),
        scratch_types=[
            pltpu.VMEM((rows_per_w, IDX_W), jnp.int32),
            pltpu.VMEM((NBUF, IDX_W), jnp.int32),
            pltpu.VMEM((NBUF, IDX_W, IDX_W), jnp.float32),
            pltpu.VMEM((NBUF, IDX_W, IDX_W), jnp.float32),
            [pltpu.SemaphoreType.DMA] * NBUF,
            [pltpu.SemaphoreType.DMA] * NBUF,
        ],
        compiler_params=pltpu.CompilerParams(needs_layout_passes=False),
    )
    def k(tok_hbm, t2_hbm, out_hbm, idx_all, idx2, rb, wb, gsems, wsems):
        wid = _wid()
        w_row0 = wid * rows_per_w
        io16 = lax.iota(jnp.int32, 16)

        pltpu.sync_copy(tok_hbm.at[pl.ds(w_row0, rows_per_w)], idx_all)

        def fire(g, b):
            for l0 in range(8):
                vv = idx_all[g, pl.ds(l0 * 16, 16)]
                idx2[b, pl.ds(l0 * 16, 16)] = lax.shift_right_logical(vv, 1)
            pltpu.async_copy(t2_hbm.at[idx2.at[b]], rb.at[b], gsems[b])

        def wait_write(g, b):
            pltpu.make_async_copy(
                wb.at[b],
                out_hbm.at[pl.ds((w_row0 + g) * IDX_W, IDX_W)],
                wsems[b],
            ).wait()

        def drain_store(g, b):
            pltpu.make_async_copy(t2_hbm.at[idx2.at[b]], rb.at[b], gsems[b]).wait()

            @pl.when(g >= NBUF)
            def _():
                wait_write(g - NBUF, b)

            for r0 in range(8):
                iv = idx_all[g, pl.ds(r0 * 16, 16)]
                rows = io16 + (r0 * 16)
                cs = (iv & 1) * 64
                for d in range(64):
                    vv = plsc.load_gather(rb.at[b], [rows, cs + d])
                    plsc.store_scatter(wb.at[b], [rows, io16 * 0 + d], vv)
            pltpu.async_copy(
                wb.at[b],
                out_hbm.at[pl.ds((w_row0 + g) * IDX_W, IDX_W)],
                wsems[b],
            )

        for b in range(NBUF):
            fire(b, b)

        def body(o, carry):
            for b in range(NBUF):
                g = o * NBUF + b

                @pl.when(g < rows_per_w)
                def _():
                    drain_store(g, b)

                    @pl.when(g + NBUF < rows_per_w)
                    def _():
                        fire(g + NBUF, b)

            return carry

        lax.fori_loop(0, (rows_per_w + NBUF - 1) // NBUF, body, 0)

        for b in range(NBUF):
            wait_write(0, b)

    return k(tok, t2)


def kernel(token_ids, embedding):
    b, s = token_ids.shape
    v, dim = embedding.shape
    tok = token_ids.reshape(-1, IDX_W).astype(jnp.int32)
    e_t = jnp.swapaxes(embedding, 0, 1)
    full = (v // IDX_W) * IDX_W
    tail2 = embedding[full:].reshape(-1, IDX_W)
    t2 = _sc_relayout(e_t, tail2, v=v)
    out = _sc_gather(tok, t2, n_rows=tok.shape[0])
    return out[:, :dim].reshape(b, s, dim)


# speedup vs baseline: 1.2180x; 1.2180x over previous
import functools

import jax
import jax.numpy as jnp
from jax import lax
from jax.experimental import pallas as pl
from jax.experimental.pallas import tpu as pltpu
from jax.experimental.pallas import tpu_sc as plsc

NC = 2
NS = 16
NW = NC * NS

IDX_W = 128
NBUF = 2
RBUF = 2


def _wid():
    return lax.axis_index("s") * NC + lax.axis_index("c")


@functools.partial(jax.jit, static_argnames=("v",))
def _sc_relayout(e_t, tail2, *, v):
    n_blocks = v // IDX_W
    per_tile = n_blocks // NW
    n_extra = n_blocks - per_tile * NW
    max_nb = per_tile + 1

    @functools.partial(
        pl.kernel,
        mesh=plsc.VectorSubcoreMesh(core_axis_name="c", subcore_axis_name="s"),
        out_type=jax.ShapeDtypeStruct((v // 2, IDX_W), jnp.float32),
        scratch_types=[
            pltpu.VMEM((RBUF, 64, IDX_W), jnp.float32),
            pltpu.VMEM((RBUF, 64, IDX_W), jnp.float32),
            [pltpu.SemaphoreType.DMA] * RBUF,
            [pltpu.SemaphoreType.DMA] * RBUF,
        ],
        compiler_params=pltpu.CompilerParams(needs_layout_passes=False),
    )
    def k(et_hbm, tail_hbm, t2_hbm, inb, outb, rsems, wsems):
        wid = _wid()
        nb_tile = jnp.where(wid < n_extra, per_tile + 1, per_tile)
        rowv = lax.iota(jnp.int32, 16) // 2
        colv = (lax.iota(jnp.int32, 16) % 2) * 64

        def blk_of(g):
            return g * NW + wid

        def fire_read(g, b):
            pltpu.async_copy(
                et_hbm.at[:, pl.ds(blk_of(g) * IDX_W, IDX_W)],
                inb.at[b],
                rsems[b],
            )

        def wait_read(g, b):
            pltpu.make_async_copy(
                et_hbm.at[:, pl.ds(blk_of(g) * IDX_W, IDX_W)],
                inb.at[b],
                rsems[b],
            ).wait()

        def transpose(b):
            for d in range(64):
                vs = [inb[b, d, pl.ds(l0 * 16, 16)] for l0 in range(8)]
                for l0 in range(8):
                    plsc.store_scatter(
                        outb.at[b], [rowv + l0 * 8, colv + d], vs[l0]
                    )

        def fire_write(g, b):
            pltpu.async_copy(
                outb.at[b],
                t2_hbm.at[pl.ds(blk_of(g) * 64, 64)],
                wsems[b],
            )

        def wait_write(g, b):
            pltpu.make_async_copy(
                outb.at[b],
                t2_hbm.at[pl.ds(blk_of(g) * 64, 64)],
                wsems[b],
            ).wait()

        for b in range(RBUF):
            fire_read(b, b)

        def body(o, carry):
            for b in range(RBUF):
                g = o * RBUF + b

                @pl.when(g < nb_tile)
                def _():
                    wait_read(g, b)

                    @pl.when(g >= RBUF)
                    def _():
                        wait_write(g - RBUF, b)

                    transpose(b)
                    fire_write(g, b)

                    @pl.when(g + RBUF < nb_tile)
                    def _():
                        fire_read(g + RBUF, b)

            return carry

        lax.fori_loop(0, (max_nb + RBUF - 1) // RBUF, body, 0)

        for b in range(RBUF):
            wait_write(0, b)

        @pl.when(wid == n_extra)
        def _():
            pltpu.sync_copy(tail_hbm, inb.at[0, pl.ds(0, 32)])
            pltpu.sync_copy(
                inb.at[0, pl.ds(0, 32)],
                t2_hbm.at[pl.ds(n_blocks * 64, 32)],
            )

    return k(e_t, tail2)


@functools.partial(jax.jit, static_argnames=("n_rows",))
def _sc_gather(tok, t2, *, n_rows):
    rows_per_w = n_rows // NW

    @functools.partial(
        pl.kernel,
        mesh=plsc.VectorSubcoreMesh(core_axis_name="c", subcore_axis_name="s"),
        out_type=jax.ShapeDtypeStruct((n_rows * IDX_W, IDX_W), jnp.float32),
        scratch_types=[
            pltpu.VMEM((rows_per_w, IDX_W), jnp.int32),
            pltpu.VMEM((NBUF, IDX_W), jnp.int32),
            pltpu.VMEM((NBUF, IDX_W, IDX_W), jnp.float32),
            pltpu.VMEM((NBUF, IDX_W, IDX_W), jnp.float32),
            [pltpu.SemaphoreType.DMA] * NBUF,
            [pltpu.SemaphoreType.DMA] * NBUF,
        ],
        compiler_params=pltpu.CompilerParams(needs_layout_passes=False),
    )
    def k(tok_hbm, t2_hbm, out_hbm, idx_all, idx2, rb, wb, gsems, wsems):
        wid = _wid()
        w_row0 = wid * rows_per_w
        io16 = lax.iota(jnp.int32, 16)

        pltpu.sync_copy(tok_hbm.at[pl.ds(w_row0, rows_per_w)], idx_all)

        def fire(g, b):
            for l0 in range(8):
                vv = idx_all[g, pl.ds(l0 * 16, 16)]
                idx2[b, pl.ds(l0 * 16, 16)] = lax.shift_right_logical(vv, 1)
            pltpu.async_copy(t2_hbm.at[idx2.at[b]], rb.at[b], gsems[b])

        def wait_write(g, b):
            pltpu.make_async_copy(
                wb.at[b],
                out_hbm.at[pl.ds((w_row0 + g) * IDX_W, IDX_W)],
                wsems[b],
            ).wait()

        def drain_store(g, b):
            pltpu.make_async_copy(t2_hbm.at[idx2.at[b]], rb.at[b], gsems[b]).wait()

            @pl.when(g >= NBUF)
            def _():
                wait_write(g - NBUF, b)

            for r0 in range(8):
                iv = idx_all[g, pl.ds(r0 * 16, 16)]
                rows = io16 + (r0 * 16)
                cs = (iv & 1) * 64
                zz = io16 * 0
                for d0 in range(0, 64, 8):
                    vs = [
                        plsc.load_gather(rb.at[b], [rows, cs + (d0 + j)])
                        for j in range(8)
                    ]
                    for j in range(8):
                        plsc.store_scatter(
                            wb.at[b], [rows, zz + (d0 + j)], vs[j]
                        )
            pltpu.async_copy(
                wb.at[b],
                out_hbm.at[pl.ds((w_row0 + g) * IDX_W, IDX_W)],
                wsems[b],
            )

        for b in range(NBUF):
            fire(b, b)

        def body(o, carry):
            for b in range(NBUF):
                g = o * NBUF + b

                @pl.when(g < rows_per_w)
                def _():
                    drain_store(g, b)

                    @pl.when(g + NBUF < rows_per_w)
                    def _():
                        fire(g + NBUF, b)

            return carry

        lax.fori_loop(0, (rows_per_w + NBUF - 1) // NBUF, body, 0)

        for b in range(NBUF):
            wait_write(0, b)

    return k(tok, t2)


def kernel(token_ids, embedding):
    b, s = token_ids.shape
    v, dim = embedding.shape
    tok = token_ids.reshape(-1, IDX_W).astype(jnp.int32)
    e_t = jnp.swapaxes(embedding, 0, 1)
    full = (v // IDX_W) * IDX_W
    tail2 = embedding[full:].reshape(-1, IDX_W)
    t2 = _sc_relayout(e_t, tail2, v=v)
    out = _sc_gather(tok, t2, n_rows=tok.shape[0])
    return out[:, :dim].reshape(b, s, dim)


# speedup vs baseline: 3.2532x; 2.6708x over previous
import functools

import jax
import jax.numpy as jnp
from jax import lax
from jax.experimental import pallas as pl
from jax.experimental.pallas import tpu as pltpu
from jax.experimental.pallas import tpu_sc as plsc

NC = 2
NS = 16
NW = NC * NS

IDX_W = 128
ROWS_PER_CHUNK = 1
NBUF = 5


@functools.partial(jax.jit, static_argnames=("n_rows", "dim"))
def _sc_gather(tok, table, *, n_rows, dim):
    chunk = ROWS_PER_CHUNK * IDX_W
    rows_per_w = n_rows // NW
    chunks_per_w = rows_per_w // ROWS_PER_CHUNK
    steady = chunks_per_w - NBUF
    assert steady % NBUF == 0
    pad_dim = table.shape[-1]

    @functools.partial(
        pl.kernel,
        mesh=plsc.VectorSubcoreMesh(core_axis_name="c", subcore_axis_name="s"),
        out_type=jax.ShapeDtypeStruct((n_rows * IDX_W, pad_dim), jnp.float32),
        scratch_types=[
            pltpu.VMEM((rows_per_w, IDX_W), jnp.int32),
            pltpu.VMEM((NBUF, chunk, pad_dim), jnp.float32),
            [pltpu.SemaphoreType.DMA] * NBUF,
        ],
    )
    def k(tok_hbm, table_hbm, out_hbm, idx_all, rb, gsems):
        wid = lax.axis_index("s") * NC + lax.axis_index("c")
        w_row0 = wid * rows_per_w

        pltpu.sync_copy(tok_hbm.at[pl.ds(w_row0, rows_per_w)], idx_all)

        def fire(g, b):
            for j in range(ROWS_PER_CHUNK):
                pltpu.async_copy(
                    table_hbm.at[idx_all.at[g * ROWS_PER_CHUNK + j]],
                    rb.at[b].at[pl.ds(j * IDX_W, IDX_W)],
                    gsems[b],
                )

        def drain_store(g, b):
            for j in range(ROWS_PER_CHUNK):
                pltpu.make_async_copy(
                    table_hbm.at[idx_all.at[g * ROWS_PER_CHUNK + j]],
                    rb.at[b].at[pl.ds(j * IDX_W, IDX_W)],
                    gsems[b],
                ).wait()
            out0 = (w_row0 + g * ROWS_PER_CHUNK) * IDX_W
            pltpu.sync_copy(rb.at[b], out_hbm.at[pl.ds(out0, chunk)])

        for b in range(NBUF):
            fire(b, b)

        def body(o, carry):
            for b in range(NBUF):
                g = o * NBUF + b
                drain_store(g, b)
                fire(g + NBUF, b)
            return carry

        lax.fori_loop(0, steady // NBUF, body, 0)

        for b in range(NBUF):
            drain_store(steady + b, b)

    return k(tok, table)


def kernel(token_ids, embedding):
    b, s = token_ids.shape
    v, dim = embedding.shape
    tok = token_ids.reshape(-1, IDX_W).astype(jnp.int32)
    t_pad = jnp.pad(embedding, ((0, 0), (0, 128 - dim)))
    out = _sc_gather(tok, t_pad, n_rows=tok.shape[0], dim=dim)
    return out[:, :dim].reshape(b, s, dim)
